# Initial kernel scaffold; baseline (speedup 1.0000x reference)
import jax
import jax.numpy as jnp
from jax.experimental import pallas as pl


def kernel(inputs, word_table, pos_table, depl_table, W1, b1, W2, b2, W3, b3):
    raise NotImplementedError("write your pallas kernel here")


# speedup vs baseline: 1.1592x; 1.1592x over previous
import functools

import jax
import jax.numpy as jnp
from jax import lax
from jax.experimental import pallas as pl
from jax.experimental.pallas import tpu as pltpu
from jax.experimental.pallas import tpu_sc as plsc

B = 16384
WORD_D, POS_D, DEPL_D = 64, 32, 32
N_WORD, N_POS, N_DEPL = 20, 20, 12
IN_SIZE = N_WORD * WORD_D + N_POS * POS_D + N_DEPL * DEPL_D
H1, H2, OUT = 512, 256, 128
POS_COL = N_WORD * WORD_D
DEPL_COL = POS_COL + N_POS * POS_D

NW = 32
BPW = B // NW
CH = 128
NCH = BPW // CH


def _sc_gather(idxT, word_table, pos_table, depl_table):
    mesh = plsc.VectorSubcoreMesh(core_axis_name="c", subcore_axis_name="s")

    @functools.partial(
        pl.kernel,
        mesh=mesh,
        out_type=jax.ShapeDtypeStruct((B, IN_SIZE), jnp.float32),
        scratch_types=[
            pltpu.VMEM((52, BPW), jnp.int32),
            pltpu.VMEM((CH, WORD_D), jnp.float32),
            pltpu.VMEM((CH, POS_D), jnp.float32),
            pltpu.SemaphoreType.DMA,
        ],
    )
    def k(idxT_hbm, word_hbm, pos_hbm, depl_hbm, out_hbm, idx_v, rows_w, rows_p, sem):
        wid = lax.axis_index("s") * 2 + lax.axis_index("c")
        base = wid * BPW
        pltpu.sync_copy(idxT_hbm.at[:, pl.ds(base, BPW)], idx_v)

        def word_body(t, carry):
            j = t // NCH
            c = t % NCH
            idx = idx_v.at[j, pl.ds(c * CH, CH)]
            pltpu.async_copy(word_hbm.at[idx], rows_w, sem).wait()
            pltpu.sync_copy(
                rows_w,
                out_hbm.at[pl.ds(base + c * CH, CH), pl.ds(j * WORD_D, WORD_D)],
            )
            return carry

        lax.fori_loop(0, N_WORD * NCH, word_body, 0)

        def pos_body(t, carry):
            j = t // NCH
            c = t % NCH
            idx = idx_v.at[N_WORD + j, pl.ds(c * CH, CH)]
            pltpu.async_copy(pos_hbm.at[idx], rows_p, sem).wait()
            pltpu.sync_copy(
                rows_p,
                out_hbm.at[pl.ds(base + c * CH, CH), pl.ds(POS_COL + j * POS_D, POS_D)],
            )
            return carry

        lax.fori_loop(0, N_POS * NCH, pos_body, 0)

        def depl_body(t, carry):
            j = t // NCH
            c = t % NCH
            idx = idx_v.at[N_WORD + N_POS + j, pl.ds(c * CH, CH)]
            pltpu.async_copy(depl_hbm.at[idx], rows_p, sem).wait()
            pltpu.sync_copy(
                rows_p,
                out_hbm.at[pl.ds(base + c * CH, CH), pl.ds(DEPL_COL + j * DEPL_D, DEPL_D)],
            )
            return carry

        lax.fori_loop(0, N_DEPL * NCH, depl_body, 0)

    return k(idxT, word_table, pos_table, depl_table)


def _mlp_body(embs_ref, w1_ref, b1_ref, w2_ref, b2_ref, w3_ref, b3_ref, out_ref):
    h = jnp.dot(embs_ref[...], w1_ref[...], preferred_element_type=jnp.float32)
    h = h + b1_ref[...]
    h = jnp.where(h >= 0, h, 0.2 * h)
    h = jnp.dot(h, w2_ref[...], preferred_element_type=jnp.float32) + b2_ref[...]
    h = jnp.where(h >= 0, h, 0.2 * h)
    out_ref[...] = jnp.dot(h, w3_ref[...], preferred_element_type=jnp.float32) + b3_ref[...]


def _tc_mlp(embs, W1, b1, W2, b2, W3, b3):
    BB = 512
    return pl.pallas_call(
        _mlp_body,
        grid=(B // BB,),
        in_specs=[
            pl.BlockSpec((BB, IN_SIZE), lambda i: (i, 0)),
            pl.BlockSpec((IN_SIZE, H1), lambda i: (0, 0)),
            pl.BlockSpec((1, H1), lambda i: (0, 0)),
            pl.BlockSpec((H1, H2), lambda i: (0, 0)),
            pl.BlockSpec((1, H2), lambda i: (0, 0)),
            pl.BlockSpec((H2, OUT), lambda i: (0, 0)),
            pl.BlockSpec((1, OUT), lambda i: (0, 0)),
        ],
        out_specs=pl.BlockSpec((BB, OUT), lambda i: (i, 0)),
        out_shape=jax.ShapeDtypeStruct((B, OUT), jnp.float32),
    )(embs, W1, b1.reshape(1, H1), W2, b2.reshape(1, H2), W3, b3.reshape(1, OUT))


def kernel(inputs, word_table, pos_table, depl_table, W1, b1, W2, b2, W3, b3):
    word_e = jnp.take(word_table, inputs[:, 0:20], axis=0).reshape(B, N_WORD * WORD_D)
    pos_e = jnp.take(pos_table, inputs[:, 20:40], axis=0).reshape(B, N_POS * POS_D)
    depl_e = jnp.take(depl_table, inputs[:, 40:52], axis=0).reshape(B, N_DEPL * DEPL_D)
    embs = jnp.concatenate([word_e, pos_e, depl_e], axis=-1)
    return _tc_mlp(embs, W1, b1, W2, b2, W3, b3)


# speedup vs baseline: 2.4420x; 2.1066x over previous
import functools

import jax
import jax.numpy as jnp
from jax import lax
from jax.experimental import pallas as pl
from jax.experimental.pallas import tpu as pltpu
from jax.experimental.pallas import tpu_sc as plsc

B = 16384
WORD_D, POS_D, DEPL_D = 64, 32, 32
N_WORD, N_POS, N_DEPL = 20, 20, 12
IN_SIZE = N_WORD * WORD_D + N_POS * POS_D + N_DEPL * DEPL_D
H1, H2, OUT = 512, 256, 128

NW = 32
BPW = B // NW
CH = 128
NCH = BPW // CH


def _sc_gather(idxT, word_table, pos_table, depl_table):
    mesh = plsc.VectorSubcoreMesh(core_axis_name="c", subcore_axis_name="s")

    @functools.partial(
        pl.kernel,
        mesh=mesh,
        compiler_params=pltpu.CompilerParams(use_tc_tiling_on_sc=False),
        out_type=[
            jax.ShapeDtypeStruct((N_WORD, B, WORD_D), jnp.float32),
            jax.ShapeDtypeStruct((N_POS, B, POS_D), jnp.float32),
            jax.ShapeDtypeStruct((N_DEPL, B, DEPL_D), jnp.float32),
        ],
        scratch_types=[
            pltpu.VMEM((52, BPW), jnp.int32),
            pltpu.VMEM((CH, WORD_D), jnp.float32),
            pltpu.VMEM((CH, POS_D), jnp.float32),
            pltpu.SemaphoreType.DMA,
        ],
    )
    def k(idxT_hbm, word_hbm, pos_hbm, depl_hbm,
          wout_hbm, pout_hbm, dout_hbm, idx_v, buf64, buf32, sem):
        wid = lax.axis_index("s") * 2 + lax.axis_index("c")
        base = wid * BPW
        pltpu.sync_copy(idxT_hbm.at[:, pl.ds(base, BPW)], idx_v)

        def word_body(t, carry):
            j = t // NCH
            c = t % NCH
            row = base + c * CH
            pltpu.async_copy(
                word_hbm.at[idx_v.at[j, pl.ds(c * CH, CH)]], buf64, sem).wait()
            pltpu.sync_copy(buf64, wout_hbm.at[j, pl.ds(row, CH), :])
            return carry

        lax.fori_loop(0, N_WORD * NCH, word_body, 0)

        def pos_body(t, carry):
            j = t // NCH
            c = t % NCH
            row = base + c * CH
            pltpu.async_copy(
                pos_hbm.at[idx_v.at[N_WORD + j, pl.ds(c * CH, CH)]], buf32, sem).wait()
            pltpu.sync_copy(buf32, pout_hbm.at[j, pl.ds(row, CH), :])
            return carry

        lax.fori_loop(0, N_POS * NCH, pos_body, 0)

        def depl_body(t, carry):
            j = t // NCH
            c = t % NCH
            row = base + c * CH
            pltpu.async_copy(
                depl_hbm.at[idx_v.at[N_WORD + N_POS + j, pl.ds(c * CH, CH)]], buf32, sem).wait()
            pltpu.sync_copy(buf32, dout_hbm.at[j, pl.ds(row, CH), :])
            return carry

        lax.fori_loop(0, N_DEPL * NCH, depl_body, 0)

    return k(idxT, word_table, pos_table, depl_table)


def _mlp_body(word_ref, pos_ref, depl_ref,
              w1_ref, b1_ref, w2_ref, b2_ref, w3_ref, b3_ref, out_ref, embs):
    for j in range(N_WORD):
        embs[:, j * WORD_D:(j + 1) * WORD_D] = word_ref[j]
    c0 = N_WORD * WORD_D
    for j in range(N_POS):
        embs[:, c0 + j * POS_D:c0 + (j + 1) * POS_D] = pos_ref[j]
    c0 = N_WORD * WORD_D + N_POS * POS_D
    for j in range(N_DEPL):
        embs[:, c0 + j * DEPL_D:c0 + (j + 1) * DEPL_D] = depl_ref[j]
    h = jnp.dot(embs[...], w1_ref[...], preferred_element_type=jnp.float32)
    h = h + b1_ref[...]
    h = jnp.where(h >= 0, h, 0.2 * h)
    h = jnp.dot(h, w2_ref[...], preferred_element_type=jnp.float32) + b2_ref[...]
    h = jnp.where(h >= 0, h, 0.2 * h)
    out_ref[...] = jnp.dot(h, w3_ref[...], preferred_element_type=jnp.float32) + b3_ref[...]


def _tc_mlp(word_sm, pos_sm, depl_sm, W1, b1, W2, b2, W3, b3):
    BB = 512
    return pl.pallas_call(
        _mlp_body,
        grid=(B // BB,),
        in_specs=[
            pl.BlockSpec((N_WORD, BB, WORD_D), lambda i: (0, i, 0)),
            pl.BlockSpec((N_POS, BB, POS_D), lambda i: (0, i, 0)),
            pl.BlockSpec((N_DEPL, BB, DEPL_D), lambda i: (0, i, 0)),
            pl.BlockSpec((IN_SIZE, H1), lambda i: (0, 0)),
            pl.BlockSpec((1, H1), lambda i: (0, 0)),
            pl.BlockSpec((H1, H2), lambda i: (0, 0)),
            pl.BlockSpec((1, H2), lambda i: (0, 0)),
            pl.BlockSpec((H2, OUT), lambda i: (0, 0)),
            pl.BlockSpec((1, OUT), lambda i: (0, 0)),
        ],
        out_specs=pl.BlockSpec((BB, OUT), lambda i: (i, 0)),
        out_shape=jax.ShapeDtypeStruct((B, OUT), jnp.float32),
        scratch_shapes=[pltpu.VMEM((BB, IN_SIZE), jnp.float32)],
    )(word_sm, pos_sm, depl_sm,
      W1, b1.reshape(1, H1), W2, b2.reshape(1, H2), W3, b3.reshape(1, OUT))


def kernel(inputs, word_table, pos_table, depl_table, W1, b1, W2, b2, W3, b3):
    idxT = inputs.astype(jnp.int32).T
    word_sm, pos_sm, depl_sm = _sc_gather(idxT, word_table, pos_table, depl_table)
    return _tc_mlp(word_sm, pos_sm, depl_sm, W1, b1, W2, b2, W3, b3)
